# Initial kernel scaffold; baseline (speedup 1.0000x reference)
import jax
import jax.numpy as jnp
from jax.experimental import pallas as pl


def kernel(sources, destinations, timestamps, edge_features, memory, w_time, b_time, W_msg, b_msg, W_i, W_h, b_i, b_h, W_fc, b_fc):
    raise NotImplementedError("write your pallas kernel here")


# speedup vs baseline: 1.8464x; 1.8464x over previous
import functools

import jax
import jax.numpy as jnp
from jax import lax
from jax.experimental import pallas as pl
from jax.experimental.pallas import tpu as pltpu
from jax.experimental.pallas import tpu_sc as plsc

N_NODES = 10000
N_EDGES = 320000
MEM_DIM = 128
EDGE_DIM = 16
TIME_DIM = 128
NC = 2
NS = 16
N_TILES = NC * NS
EDGES_PER_TILE = N_EDGES // N_TILES
CHUNK = 40
N_CHUNKS = EDGES_PER_TILE // CHUNK
IDX_GROUP = 5
N_PAD = 10240
ROWS_PER_TILE = N_PAD // NS
LANES = 16
EB = 2000


def _proj_body(mem_ref, wa_ref, wb_ref, pa_ref, pb_ref):
    m = mem_ref[...]
    pa_ref[...] = jnp.dot(m, wa_ref[...], preferred_element_type=jnp.float32)
    pb_ref[...] = jnp.dot(m, wb_ref[...], preferred_element_type=jnp.float32)


def _project_tables(memory, wa, wb):
    return pl.pallas_call(
        _proj_body,
        out_shape=(
            jax.ShapeDtypeStruct((N_NODES, MEM_DIM), jnp.float32),
            jax.ShapeDtypeStruct((N_NODES, MEM_DIM), jnp.float32),
        ),
    )(memory, wa, wb)


def _edge_body(ts_ref, ef_ref, dst_ref, wt_ref, bt_ref, we_ref, wtm_ref,
               bm_ref, e_ref, cnt_ref):
    i = pl.program_id(0)
    t = ts_ref[...]
    tenc = jnp.cos(t * wt_ref[...] + bt_ref[...])
    acc = jnp.dot(ef_ref[...], we_ref[...], preferred_element_type=jnp.float32)
    acc += jnp.dot(tenc, wtm_ref[...], preferred_element_type=jnp.float32)
    e_ref[...] = acc + bm_ref[...]
    d = dst_ref[...]
    iota = lax.broadcasted_iota(jnp.int32, (EB, MEM_DIM), 1)
    oh_lo = (jnp.bitwise_and(d, 127) == iota).astype(jnp.float32)
    oh_hi = (jnp.right_shift(d, 7) == iota).astype(jnp.float32)
    part = lax.dot_general(oh_hi, oh_lo, (((0,), (0,)), ((), ())),
                           preferred_element_type=jnp.float32)

    @pl.when(i == 0)
    def _():
        cnt_ref[...] = jnp.zeros_like(cnt_ref)

    cnt_ref[...] += part


def _edge_dense(ts2, ef, dst2, w_time, b_time, we, wt, b_msg):
    grid = (N_EDGES // EB,)
    return pl.pallas_call(
        _edge_body,
        grid=grid,
        in_specs=[
            pl.BlockSpec((EB, 1), lambda i: (i, 0)),
            pl.BlockSpec((EB, EDGE_DIM), lambda i: (i, 0)),
            pl.BlockSpec((EB, 1), lambda i: (i, 0)),
            pl.BlockSpec((1, TIME_DIM), lambda i: (0, 0)),
            pl.BlockSpec((1, TIME_DIM), lambda i: (0, 0)),
            pl.BlockSpec((EDGE_DIM, MEM_DIM), lambda i: (0, 0)),
            pl.BlockSpec((TIME_DIM, MEM_DIM), lambda i: (0, 0)),
            pl.BlockSpec((1, MEM_DIM), lambda i: (0, 0)),
        ],
        out_specs=(
            pl.BlockSpec((EB, MEM_DIM), lambda i: (i, 0)),
            pl.BlockSpec((MEM_DIM, MEM_DIM), lambda i: (0, 0)),
        ),
        out_shape=(
            jax.ShapeDtypeStruct((N_EDGES, MEM_DIM), jnp.float32),
            jax.ShapeDtypeStruct((MEM_DIM, MEM_DIM), jnp.float32),
        ),
    )(ts2, ef, dst2, w_time, b_time, we, wt, b_msg)


def _sc_body(psrc, pdst, e_hbm, src3, dst3,
             agg_out,
             idx_sc, idx_dc, rs, rd, sem,
             agg_sh):
    c = lax.axis_index("c")
    s = lax.axis_index("s")
    wid = c * NS + s
    row0 = s * ROWS_PER_TILE

    def zrow(i, carry2):
        for k in range(MEM_DIM // LANES):
            rs[i, pl.ds(k * LANES, LANES)] = jnp.zeros((LANES,), jnp.float32)
        return carry2

    lax.fori_loop(0, CHUNK, zrow, 0)
    for b in range(ROWS_PER_TILE // CHUNK):
        pltpu.sync_copy(rs, agg_sh.at[pl.ds(row0 + b * CHUNK, CHUNK)])

    plsc.subcore_barrier()

    ebase = wid * EDGES_PER_TILE

    def chunk(j, carry):
        pltpu.sync_copy(src3.at[wid, j], idx_sc)
        pltpu.sync_copy(dst3.at[wid, j], idx_dc)
        cp_s = pltpu.async_copy(psrc.at[idx_sc], rs, sem)
        cp_d = pltpu.async_copy(pdst.at[idx_dc], rd, sem)
        cp_s.wait()
        cp_d.wait()

        def row1(i, carry2):
            for k in range(MEM_DIM // LANES):
                sl = pl.ds(k * LANES, LANES)
                rs[i, sl] += rd[i, sl]
            return carry2

        lax.fori_loop(0, CHUNK, row1, 0)

        cp_e = pltpu.async_copy(e_hbm.at[pl.ds(ebase + j * CHUNK, CHUNK)], rd, sem)
        cp_e.wait()

        def row2(i, carry2):
            for k in range(MEM_DIM // LANES):
                sl = pl.ds(k * LANES, LANES)
                rs[i, sl] = jnp.maximum(rs[i, sl] + rd[i, sl], 0.0)
            return carry2

        lax.fori_loop(0, CHUNK, row2, 0)

        pltpu.sync_copy(rs, agg_sh.at[idx_dc], add=True)
        return carry

    lax.fori_loop(0, N_CHUNKS, chunk, 0)
    plsc.subcore_barrier()

    for b in range(ROWS_PER_TILE // CHUNK):
        r0 = row0 + b * CHUNK
        pltpu.sync_copy(agg_sh.at[pl.ds(r0, CHUNK)], rs)
        pltpu.sync_copy(rs, agg_out.at[c, pl.ds(r0, CHUNK)])


def _sc_aggregate(psrc, pdst, e, src3, dst3):
    mesh = plsc.VectorSubcoreMesh(core_axis_name="c", subcore_axis_name="s")
    f = pl.kernel(
        _sc_body,
        out_type=jax.ShapeDtypeStruct((NC, N_PAD, MEM_DIM), jnp.float32),
        mesh=mesh,
        scratch_types=[
            pltpu.VMEM((CHUNK,), jnp.int32),
            pltpu.VMEM((CHUNK,), jnp.int32),
            pltpu.VMEM((CHUNK, MEM_DIM), jnp.float32),
            pltpu.VMEM((CHUNK, MEM_DIM), jnp.float32),
            pltpu.SemaphoreType.DMA,
            pltpu.VMEM_SHARED((N_PAD, MEM_DIM), jnp.float32),
        ],
    )
    return f(psrc, pdst, e, src3, dst3)


def _head_body(agg_ref, cnt_ref, mem_ref, wi_ref, wh_ref, bi_ref, bh_ref,
               wfc_ref, bfc_ref, out_ref):
    a = agg_ref[0, :N_NODES] + agg_ref[1, :N_NODES]
    agg = a / jnp.maximum(cnt_ref[...], 1.0)
    mem = mem_ref[...]
    gi = jnp.dot(agg, wi_ref[...], preferred_element_type=jnp.float32) + bi_ref[...]
    gh = jnp.dot(mem, wh_ref[...], preferred_element_type=jnp.float32) + bh_ref[...]
    r = jax.nn.sigmoid(gi[:, :MEM_DIM] + gh[:, :MEM_DIM])
    z = jax.nn.sigmoid(gi[:, MEM_DIM:2 * MEM_DIM] + gh[:, MEM_DIM:2 * MEM_DIM])
    n = jnp.tanh(gi[:, 2 * MEM_DIM:] + r * gh[:, 2 * MEM_DIM:])
    upd = (1.0 - z) * n + z * mem
    tot = jnp.sum(upd, axis=0, keepdims=True) - upd[0:1, :]
    feat = jnp.tanh(tot / (N_NODES - 1.0))
    logits = jnp.dot(feat, wfc_ref[...], preferred_element_type=jnp.float32) + bfc_ref[...]
    m = jnp.max(logits, axis=1, keepdims=True)
    ex = jnp.exp(logits - m)
    out_ref[...] = ex / jnp.sum(ex, axis=1, keepdims=True)


def _gru_head(agg2, counts, memory, wi, wh, bi, bh, wfc, bfc):
    return pl.pallas_call(
        _head_body,
        out_shape=jax.ShapeDtypeStruct((1, 2), jnp.float32),
    )(agg2, counts, memory, wi, wh, bi, bh, wfc, bfc)


def kernel(sources, destinations, timestamps, edge_features, memory,
           w_time, b_time, W_msg, b_msg, W_i, W_h, b_i, b_h, W_fc, b_fc):
    wa = W_msg[:MEM_DIM]
    wb = W_msg[MEM_DIM:2 * MEM_DIM]
    we = W_msg[2 * MEM_DIM:2 * MEM_DIM + EDGE_DIM]
    wt = W_msg[2 * MEM_DIM + EDGE_DIM:]

    psrc, pdst = _project_tables(memory, wa, wb)
    e, cnt2d = _edge_dense(
        timestamps.reshape(N_EDGES, 1),
        edge_features,
        destinations.reshape(N_EDGES, 1),
        w_time.reshape(1, TIME_DIM),
        b_time.reshape(1, TIME_DIM),
        we, wt,
        b_msg.reshape(1, MEM_DIM),
    )
    counts = cnt2d.reshape(MEM_DIM * MEM_DIM, 1)[:N_NODES]

    src3 = sources.reshape(N_TILES, N_CHUNKS, CHUNK)
    dst3 = destinations.reshape(N_TILES, N_CHUNKS, CHUNK)

    agg2 = _sc_aggregate(psrc, pdst, e, src3, dst3)

    return _gru_head(
        agg2, counts, memory, W_i, W_h,
        b_i.reshape(1, 3 * MEM_DIM), b_h.reshape(1, 3 * MEM_DIM),
        W_fc, b_fc.reshape(1, 2),
    )


# speedup vs baseline: 2.3175x; 1.2551x over previous
import functools

import jax
import jax.numpy as jnp
from jax import lax
from jax.experimental import pallas as pl
from jax.experimental.pallas import tpu as pltpu
from jax.experimental.pallas import tpu_sc as plsc

N_NODES = 10000
N_EDGES = 320000
MEM_DIM = 128
EDGE_DIM = 16
TIME_DIM = 128
NC = 2
NS = 16
N_TILES = NC * NS
EDGES_PER_TILE = N_EDGES // N_TILES
CHUNK = 80
N_CHUNKS = EDGES_PER_TILE // CHUNK
IDX_GROUP = 5
N_PAD = 10240
ROWS_PER_TILE = N_PAD // NS
LANES = 16
EB = 2000


def _proj_body(mem_ref, wa_ref, wb_ref, pa_ref, pb_ref):
    m = mem_ref[...]
    pa_ref[...] = jnp.dot(m, wa_ref[...], preferred_element_type=jnp.float32)
    pb_ref[...] = jnp.dot(m, wb_ref[...], preferred_element_type=jnp.float32)


def _project_tables(memory, wa, wb):
    return pl.pallas_call(
        _proj_body,
        out_shape=(
            jax.ShapeDtypeStruct((N_NODES, MEM_DIM), jnp.float32),
            jax.ShapeDtypeStruct((N_NODES, MEM_DIM), jnp.float32),
        ),
    )(memory, wa, wb)


def _edge_body(ts_ref, ef_ref, dst_ref, wt_ref, bt_ref, we_ref, wtm_ref,
               bm_ref, e_ref, cnt_ref):
    i = pl.program_id(0)
    t = ts_ref[...]
    tenc = jnp.cos(t * wt_ref[...] + bt_ref[...])
    acc = jnp.dot(ef_ref[...], we_ref[...], preferred_element_type=jnp.float32)
    acc += jnp.dot(tenc, wtm_ref[...], preferred_element_type=jnp.float32)
    e_ref[...] = acc + bm_ref[...]
    d = dst_ref[...]
    iota = lax.broadcasted_iota(jnp.int32, (EB, MEM_DIM), 1)
    oh_lo = (jnp.bitwise_and(d, 127) == iota).astype(jnp.float32)
    oh_hi = (jnp.right_shift(d, 7) == iota).astype(jnp.float32)
    part = lax.dot_general(oh_hi, oh_lo, (((0,), (0,)), ((), ())),
                           preferred_element_type=jnp.float32)

    @pl.when(i == 0)
    def _():
        cnt_ref[...] = jnp.zeros_like(cnt_ref)

    cnt_ref[...] += part


def _edge_dense(ts2, ef, dst2, w_time, b_time, we, wt, b_msg):
    grid = (N_EDGES // EB,)
    return pl.pallas_call(
        _edge_body,
        grid=grid,
        in_specs=[
            pl.BlockSpec((EB, 1), lambda i: (i, 0)),
            pl.BlockSpec((EB, EDGE_DIM), lambda i: (i, 0)),
            pl.BlockSpec((EB, 1), lambda i: (i, 0)),
            pl.BlockSpec((1, TIME_DIM), lambda i: (0, 0)),
            pl.BlockSpec((1, TIME_DIM), lambda i: (0, 0)),
            pl.BlockSpec((EDGE_DIM, MEM_DIM), lambda i: (0, 0)),
            pl.BlockSpec((TIME_DIM, MEM_DIM), lambda i: (0, 0)),
            pl.BlockSpec((1, MEM_DIM), lambda i: (0, 0)),
        ],
        out_specs=(
            pl.BlockSpec((EB, MEM_DIM), lambda i: (i, 0)),
            pl.BlockSpec((MEM_DIM, MEM_DIM), lambda i: (0, 0)),
        ),
        out_shape=(
            jax.ShapeDtypeStruct((N_EDGES, MEM_DIM), jnp.float32),
            jax.ShapeDtypeStruct((MEM_DIM, MEM_DIM), jnp.float32),
        ),
    )(ts2, ef, dst2, w_time, b_time, we, wt, b_msg)


def _sc_body(psrc, pdst, e_hbm, src3, dst3,
             agg_out,
             idx_sc, idx_dc, rs, rd, eb, sem,
             agg_sh):
    c = lax.axis_index("c")
    s = lax.axis_index("s")
    wid = c * NS + s
    row0 = s * ROWS_PER_TILE

    def zrow(i, carry2):
        for k in range(MEM_DIM // LANES):
            rs[i, pl.ds(k * LANES, LANES)] = jnp.zeros((LANES,), jnp.float32)
        return carry2

    lax.fori_loop(0, CHUNK, zrow, 0)
    for b in range(ROWS_PER_TILE // CHUNK):
        pltpu.sync_copy(rs, agg_sh.at[pl.ds(row0 + b * CHUNK, CHUNK)])

    plsc.subcore_barrier()

    ebase = wid * EDGES_PER_TILE

    def chunk(j, carry):
        pltpu.sync_copy(src3.at[wid, j], idx_sc)
        pltpu.sync_copy(dst3.at[wid, j], idx_dc)
        cp_s = pltpu.async_copy(psrc.at[idx_sc], rs, sem)
        cp_d = pltpu.async_copy(pdst.at[idx_dc], rd, sem)
        cp_e = pltpu.async_copy(e_hbm.at[pl.ds(ebase + j * CHUNK, CHUNK)], eb, sem)
        cp_s.wait()
        cp_d.wait()
        cp_e.wait()

        def row1(i, carry2):
            for k in range(MEM_DIM // LANES):
                sl = pl.ds(k * LANES, LANES)
                rs[i, sl] = jnp.maximum(rs[i, sl] + rd[i, sl] + eb[i, sl], 0.0)
            return carry2

        lax.fori_loop(0, CHUNK, row1, 0)

        pltpu.sync_copy(rs, agg_sh.at[idx_dc], add=True)
        return carry

    lax.fori_loop(0, N_CHUNKS, chunk, 0)
    plsc.subcore_barrier()

    for b in range(ROWS_PER_TILE // CHUNK):
        r0 = row0 + b * CHUNK
        pltpu.sync_copy(agg_sh.at[pl.ds(r0, CHUNK)], rs)
        pltpu.sync_copy(rs, agg_out.at[c, pl.ds(r0, CHUNK)])


def _sc_aggregate(psrc, pdst, e, src3, dst3):
    mesh = plsc.VectorSubcoreMesh(core_axis_name="c", subcore_axis_name="s")
    f = pl.kernel(
        _sc_body,
        out_type=jax.ShapeDtypeStruct((NC, N_PAD, MEM_DIM), jnp.float32),
        mesh=mesh,
        scratch_types=[
            pltpu.VMEM((CHUNK,), jnp.int32),
            pltpu.VMEM((CHUNK,), jnp.int32),
            pltpu.VMEM((CHUNK, MEM_DIM), jnp.float32),
            pltpu.VMEM((CHUNK, MEM_DIM), jnp.float32),
            pltpu.VMEM((CHUNK, MEM_DIM), jnp.float32),
            pltpu.SemaphoreType.DMA,
            pltpu.VMEM_SHARED((N_PAD, MEM_DIM), jnp.float32),
        ],
    )
    return f(psrc, pdst, e, src3, dst3)


def _head_body(agg_ref, cnt_ref, mem_ref, wi_ref, wh_ref, bi_ref, bh_ref,
               wfc_ref, bfc_ref, out_ref):
    a = agg_ref[0, :N_NODES] + agg_ref[1, :N_NODES]
    agg = a / jnp.maximum(cnt_ref[...], 1.0)
    mem = mem_ref[...]
    gi = jnp.dot(agg, wi_ref[...], preferred_element_type=jnp.float32) + bi_ref[...]
    gh = jnp.dot(mem, wh_ref[...], preferred_element_type=jnp.float32) + bh_ref[...]
    r = jax.nn.sigmoid(gi[:, :MEM_DIM] + gh[:, :MEM_DIM])
    z = jax.nn.sigmoid(gi[:, MEM_DIM:2 * MEM_DIM] + gh[:, MEM_DIM:2 * MEM_DIM])
    n = jnp.tanh(gi[:, 2 * MEM_DIM:] + r * gh[:, 2 * MEM_DIM:])
    upd = (1.0 - z) * n + z * mem
    tot = jnp.sum(upd, axis=0, keepdims=True) - upd[0:1, :]
    feat = jnp.tanh(tot / (N_NODES - 1.0))
    logits = jnp.dot(feat, wfc_ref[...], preferred_element_type=jnp.float32) + bfc_ref[...]
    m = jnp.max(logits, axis=1, keepdims=True)
    ex = jnp.exp(logits - m)
    out_ref[...] = ex / jnp.sum(ex, axis=1, keepdims=True)


def _gru_head(agg2, counts, memory, wi, wh, bi, bh, wfc, bfc):
    return pl.pallas_call(
        _head_body,
        out_shape=jax.ShapeDtypeStruct((1, 2), jnp.float32),
    )(agg2, counts, memory, wi, wh, bi, bh, wfc, bfc)


def kernel(sources, destinations, timestamps, edge_features, memory,
           w_time, b_time, W_msg, b_msg, W_i, W_h, b_i, b_h, W_fc, b_fc):
    wa = W_msg[:MEM_DIM]
    wb = W_msg[MEM_DIM:2 * MEM_DIM]
    we = W_msg[2 * MEM_DIM:2 * MEM_DIM + EDGE_DIM]
    wt = W_msg[2 * MEM_DIM + EDGE_DIM:]

    psrc, pdst = _project_tables(memory, wa, wb)
    e, cnt2d = _edge_dense(
        timestamps.reshape(N_EDGES, 1),
        edge_features,
        destinations.reshape(N_EDGES, 1),
        w_time.reshape(1, TIME_DIM),
        b_time.reshape(1, TIME_DIM),
        we, wt,
        b_msg.reshape(1, MEM_DIM),
    )
    counts = cnt2d.reshape(MEM_DIM * MEM_DIM, 1)[:N_NODES]

    src3 = sources.reshape(N_TILES, N_CHUNKS, CHUNK)
    dst3 = destinations.reshape(N_TILES, N_CHUNKS, CHUNK)

    agg2 = _sc_aggregate(psrc, pdst, e, src3, dst3)

    return _gru_head(
        agg2, counts, memory, W_i, W_h,
        b_i.reshape(1, 3 * MEM_DIM), b_h.reshape(1, 3 * MEM_DIM),
        W_fc, b_fc.reshape(1, 2),
    )


# speedup vs baseline: 2.7769x; 1.1983x over previous
import functools

import jax
import jax.numpy as jnp
from jax import lax
from jax.experimental import pallas as pl
from jax.experimental.pallas import tpu as pltpu
from jax.experimental.pallas import tpu_sc as plsc

N_NODES = 10000
N_EDGES = 320000
MEM_DIM = 128
EDGE_DIM = 16
TIME_DIM = 128
NC = 2
NS = 16
N_TILES = NC * NS
EDGES_PER_TILE = N_EDGES // N_TILES
CHUNK = 80
N_CHUNKS = EDGES_PER_TILE // CHUNK
IDX_GROUP = 5
N_PAD = 10240
ROWS_PER_TILE = N_PAD // NS
LANES = 16
EB = 2000


def _proj_body(mem_ref, wa_ref, wb_ref, pa_ref, pb_ref):
    m = mem_ref[...]
    pa_ref[...] = jnp.dot(m, wa_ref[...], preferred_element_type=jnp.float32)
    pb_ref[...] = jnp.dot(m, wb_ref[...], preferred_element_type=jnp.float32)


def _project_tables(memory, wa, wb):
    return pl.pallas_call(
        _proj_body,
        out_shape=(
            jax.ShapeDtypeStruct((N_NODES, MEM_DIM), jnp.float32),
            jax.ShapeDtypeStruct((N_NODES, MEM_DIM), jnp.float32),
        ),
    )(memory, wa, wb)


def _edge_body(ts_ref, ef_ref, dst_ref, wt_ref, bt_ref, we_ref, wtm_ref,
               bm_ref, e_ref, cnt_ref):
    i = pl.program_id(0)
    t = ts_ref[0]
    tencT = jnp.cos(wt_ref[...] * t + bt_ref[...])
    acc = lax.dot_general(tencT.astype(jnp.bfloat16), wtm_ref[...],
                          (((0,), (0,)), ((), ())),
                          preferred_element_type=jnp.float32)
    acc += jnp.dot(ef_ref[...].astype(jnp.bfloat16), we_ref[...],
                   preferred_element_type=jnp.float32)
    e_ref[...] = acc + bm_ref[...]
    d = dst_ref[0]
    iota = lax.broadcasted_iota(jnp.int32, (MEM_DIM, EB), 0)
    oh_lo = (jnp.bitwise_and(d, 127) == iota).astype(jnp.bfloat16)
    oh_hi = (jnp.right_shift(d, 7) == iota).astype(jnp.bfloat16)
    part = lax.dot_general(oh_hi, oh_lo, (((1,), (1,)), ((), ())),
                           preferred_element_type=jnp.float32)

    @pl.when(i == 0)
    def _():
        cnt_ref[...] = jnp.zeros_like(cnt_ref)

    cnt_ref[...] += part


def _edge_dense(ts2, ef, dst2, w_time, b_time, we, wt, b_msg):
    grid = (N_EDGES // EB,)
    return pl.pallas_call(
        _edge_body,
        grid=grid,
        in_specs=[
            pl.BlockSpec((1, 1, EB), lambda i: (i, 0, 0)),
            pl.BlockSpec((EB, EDGE_DIM), lambda i: (i, 0)),
            pl.BlockSpec((1, 1, EB), lambda i: (i, 0, 0)),
            pl.BlockSpec((TIME_DIM, 1), lambda i: (0, 0)),
            pl.BlockSpec((TIME_DIM, 1), lambda i: (0, 0)),
            pl.BlockSpec((EDGE_DIM, MEM_DIM), lambda i: (0, 0)),
            pl.BlockSpec((TIME_DIM, MEM_DIM), lambda i: (0, 0)),
            pl.BlockSpec((1, MEM_DIM), lambda i: (0, 0)),
        ],
        out_specs=(
            pl.BlockSpec((EB, MEM_DIM), lambda i: (i, 0)),
            pl.BlockSpec((MEM_DIM, MEM_DIM), lambda i: (0, 0)),
        ),
        out_shape=(
            jax.ShapeDtypeStruct((N_EDGES, MEM_DIM), jnp.float32),
            jax.ShapeDtypeStruct((MEM_DIM, MEM_DIM), jnp.float32),
        ),
    )(ts2, ef, dst2, w_time, b_time, we, wt, b_msg)


def _sc_body(psrc, pdst, e_hbm, src3, dst3,
             agg_out,
             idx_sc, idx_dc, rs, rd, eb, sem,
             agg_sh):
    c = lax.axis_index("c")
    s = lax.axis_index("s")
    wid = c * NS + s
    row0 = s * ROWS_PER_TILE

    def zrow(i, carry2):
        for k in range(MEM_DIM // LANES):
            rs[i, pl.ds(k * LANES, LANES)] = jnp.zeros((LANES,), jnp.float32)
        return carry2

    lax.fori_loop(0, CHUNK, zrow, 0)
    for b in range(ROWS_PER_TILE // CHUNK):
        pltpu.sync_copy(rs, agg_sh.at[pl.ds(row0 + b * CHUNK, CHUNK)])

    plsc.subcore_barrier()

    ebase = wid * EDGES_PER_TILE

    def chunk(j, carry):
        pltpu.sync_copy(src3.at[wid, j], idx_sc)
        pltpu.sync_copy(dst3.at[wid, j], idx_dc)
        cp_s = pltpu.async_copy(psrc.at[idx_sc], rs, sem)
        cp_d = pltpu.async_copy(pdst.at[idx_dc], rd, sem)
        cp_e = pltpu.async_copy(e_hbm.at[pl.ds(ebase + j * CHUNK, CHUNK)], eb, sem)
        cp_s.wait()
        cp_d.wait()
        cp_e.wait()

        def row1(i, carry2):
            for k in range(MEM_DIM // LANES):
                sl = pl.ds(k * LANES, LANES)
                rs[i, sl] = jnp.maximum(rs[i, sl] + rd[i, sl] + eb[i, sl], 0.0)
            return carry2

        lax.fori_loop(0, CHUNK, row1, 0)

        pltpu.sync_copy(rs, agg_sh.at[idx_dc], add=True)
        return carry

    lax.fori_loop(0, N_CHUNKS, chunk, 0)
    plsc.subcore_barrier()

    for b in range(ROWS_PER_TILE // CHUNK):
        r0 = row0 + b * CHUNK
        pltpu.sync_copy(agg_sh.at[pl.ds(r0, CHUNK)], rs)
        pltpu.sync_copy(rs, agg_out.at[c, pl.ds(r0, CHUNK)])


def _sc_aggregate(psrc, pdst, e, src3, dst3):
    mesh = plsc.VectorSubcoreMesh(core_axis_name="c", subcore_axis_name="s")
    f = pl.kernel(
        _sc_body,
        out_type=jax.ShapeDtypeStruct((NC, N_PAD, MEM_DIM), jnp.float32),
        mesh=mesh,
        scratch_types=[
            pltpu.VMEM((CHUNK,), jnp.int32),
            pltpu.VMEM((CHUNK,), jnp.int32),
            pltpu.VMEM((CHUNK, MEM_DIM), jnp.float32),
            pltpu.VMEM((CHUNK, MEM_DIM), jnp.float32),
            pltpu.VMEM((CHUNK, MEM_DIM), jnp.float32),
            pltpu.SemaphoreType.DMA,
            pltpu.VMEM_SHARED((N_PAD, MEM_DIM), jnp.float32),
        ],
    )
    return f(psrc, pdst, e, src3, dst3)


def _head_body(agg_ref, cnt_ref, mem_ref, wi_ref, wh_ref, bi_ref, bh_ref,
               wfc_ref, bfc_ref, out_ref):
    a = agg_ref[0, :N_NODES] + agg_ref[1, :N_NODES]
    agg = a / jnp.maximum(cnt_ref[...], 1.0)
    mem = mem_ref[...]
    gi = jnp.dot(agg, wi_ref[...], preferred_element_type=jnp.float32) + bi_ref[...]
    gh = jnp.dot(mem, wh_ref[...], preferred_element_type=jnp.float32) + bh_ref[...]
    r = jax.nn.sigmoid(gi[:, :MEM_DIM] + gh[:, :MEM_DIM])
    z = jax.nn.sigmoid(gi[:, MEM_DIM:2 * MEM_DIM] + gh[:, MEM_DIM:2 * MEM_DIM])
    n = jnp.tanh(gi[:, 2 * MEM_DIM:] + r * gh[:, 2 * MEM_DIM:])
    upd = (1.0 - z) * n + z * mem
    tot = jnp.sum(upd, axis=0, keepdims=True) - upd[0:1, :]
    feat = jnp.tanh(tot / (N_NODES - 1.0))
    logits = jnp.dot(feat, wfc_ref[...], preferred_element_type=jnp.float32) + bfc_ref[...]
    m = jnp.max(logits, axis=1, keepdims=True)
    ex = jnp.exp(logits - m)
    out_ref[...] = ex / jnp.sum(ex, axis=1, keepdims=True)


def _gru_head(agg2, counts, memory, wi, wh, bi, bh, wfc, bfc):
    return pl.pallas_call(
        _head_body,
        out_shape=jax.ShapeDtypeStruct((1, 2), jnp.float32),
    )(agg2, counts, memory, wi, wh, bi, bh, wfc, bfc)


def kernel(sources, destinations, timestamps, edge_features, memory,
           w_time, b_time, W_msg, b_msg, W_i, W_h, b_i, b_h, W_fc, b_fc):
    wa = W_msg[:MEM_DIM]
    wb = W_msg[MEM_DIM:2 * MEM_DIM]
    we = W_msg[2 * MEM_DIM:2 * MEM_DIM + EDGE_DIM]
    wt = W_msg[2 * MEM_DIM + EDGE_DIM:]

    psrc, pdst = _project_tables(memory, wa, wb)
    e, cnt2d = _edge_dense(
        timestamps.reshape(N_EDGES // EB, 1, EB),
        edge_features,
        destinations.reshape(N_EDGES // EB, 1, EB),
        w_time.reshape(TIME_DIM, 1),
        b_time.reshape(TIME_DIM, 1),
        we.astype(jnp.bfloat16), wt.astype(jnp.bfloat16),
        b_msg.reshape(1, MEM_DIM),
    )
    counts = cnt2d.reshape(MEM_DIM * MEM_DIM, 1)[:N_NODES]

    src3 = sources.reshape(N_TILES, N_CHUNKS, CHUNK)
    dst3 = destinations.reshape(N_TILES, N_CHUNKS, CHUNK)

    agg2 = _sc_aggregate(psrc, pdst, e, src3, dst3)

    return _gru_head(
        agg2, counts, memory, W_i, W_h,
        b_i.reshape(1, 3 * MEM_DIM), b_h.reshape(1, 3 * MEM_DIM),
        W_fc, b_fc.reshape(1, 2),
    )


# speedup vs baseline: 3.9905x; 1.4370x over previous
import functools

import jax
import jax.numpy as jnp
from jax import lax
from jax.experimental import pallas as pl
from jax.experimental.pallas import tpu as pltpu
from jax.experimental.pallas import tpu_sc as plsc

N_NODES = 10000
N_EDGES = 320000
MEM_DIM = 128
EDGE_DIM = 16
TIME_DIM = 128
NC = 2
NS = 16
N_TILES = NC * NS
EDGES_PER_TILE = N_EDGES // N_TILES
CHUNK = 80
N_CHUNKS = EDGES_PER_TILE // CHUNK
IDX_GROUP = 5
N_PAD = 10240
ROWS_PER_TILE = N_PAD // NS
LANES = 16
EB = 2000


def _proj_body(mem_ref, wa_ref, wb_ref, pa_ref, pb_ref):
    m = mem_ref[...]
    pa_ref[...] = jnp.dot(m, wa_ref[...], preferred_element_type=jnp.float32)
    pb_ref[...] = jnp.dot(m, wb_ref[...], preferred_element_type=jnp.float32)


def _project_tables(memory, wa, wb):
    return pl.pallas_call(
        _proj_body,
        out_shape=(
            jax.ShapeDtypeStruct((N_NODES, MEM_DIM), jnp.float32),
            jax.ShapeDtypeStruct((N_NODES, MEM_DIM), jnp.float32),
        ),
    )(memory, wa, wb)


def _edge_body(ts_ref, ef_ref, dst_ref, wt_ref, bt_ref, we_ref, wtm_ref,
               bm_ref, e_ref, cnt_ref):
    i = pl.program_id(0)
    t = ts_ref[0]
    a = wt_ref[...] * t + bt_ref[...]
    v = a * 0.15915494309189535
    r = v - jnp.round(v)
    u = r * r
    p = 6.575611642716226
    for coef in (-26.000527873747124, 60.176230338867754, -85.45116579292079,
                 64.9391722325954, -19.73920555404448, 0.999999992290297):
        p = p * u + coef
    tencT = p
    acc = lax.dot_general(tencT.astype(jnp.bfloat16), wtm_ref[...],
                          (((0,), (0,)), ((), ())),
                          preferred_element_type=jnp.float32)
    acc += jnp.dot(ef_ref[...].astype(jnp.bfloat16), we_ref[...],
                   preferred_element_type=jnp.float32)
    e_ref[...] = acc + bm_ref[...]
    d = dst_ref[0]
    iota = lax.broadcasted_iota(jnp.int32, (MEM_DIM, EB), 0)
    oh_lo = (jnp.bitwise_and(d, 127) == iota).astype(jnp.bfloat16)
    oh_hi = (jnp.right_shift(d, 7) == iota).astype(jnp.bfloat16)
    part = lax.dot_general(oh_hi, oh_lo, (((1,), (1,)), ((), ())),
                           preferred_element_type=jnp.float32)

    @pl.when(i == 0)
    def _():
        cnt_ref[...] = jnp.zeros_like(cnt_ref)

    cnt_ref[...] += part


def _edge_dense(ts2, ef, dst2, w_time, b_time, we, wt, b_msg):
    grid = (N_EDGES // EB,)
    return pl.pallas_call(
        _edge_body,
        grid=grid,
        in_specs=[
            pl.BlockSpec((1, 1, EB), lambda i: (i, 0, 0)),
            pl.BlockSpec((EB, EDGE_DIM), lambda i: (i, 0)),
            pl.BlockSpec((1, 1, EB), lambda i: (i, 0, 0)),
            pl.BlockSpec((TIME_DIM, 1), lambda i: (0, 0)),
            pl.BlockSpec((TIME_DIM, 1), lambda i: (0, 0)),
            pl.BlockSpec((EDGE_DIM, MEM_DIM), lambda i: (0, 0)),
            pl.BlockSpec((TIME_DIM, MEM_DIM), lambda i: (0, 0)),
            pl.BlockSpec((1, MEM_DIM), lambda i: (0, 0)),
        ],
        out_specs=(
            pl.BlockSpec((EB, MEM_DIM), lambda i: (i, 0)),
            pl.BlockSpec((MEM_DIM, MEM_DIM), lambda i: (0, 0)),
        ),
        out_shape=(
            jax.ShapeDtypeStruct((N_EDGES, MEM_DIM), jnp.float32),
            jax.ShapeDtypeStruct((MEM_DIM, MEM_DIM), jnp.float32),
        ),
    )(ts2, ef, dst2, w_time, b_time, we, wt, b_msg)


def _sc_body(psrc, pdst, e_hbm, src3, dst3,
             agg_out,
             idx_sc, idx_dc, rs, rd, eb, sem,
             agg_sh):
    c = lax.axis_index("c")
    s = lax.axis_index("s")
    wid = c * NS + s
    row0 = s * ROWS_PER_TILE

    def zrow(i, carry2):
        for k in range(MEM_DIM // LANES):
            rs[i, pl.ds(k * LANES, LANES)] = jnp.zeros((LANES,), jnp.float32)
        return carry2

    lax.fori_loop(0, CHUNK, zrow, 0)
    for b in range(ROWS_PER_TILE // CHUNK):
        pltpu.sync_copy(rs, agg_sh.at[pl.ds(row0 + b * CHUNK, CHUNK)])

    plsc.subcore_barrier()

    ebase = wid * EDGES_PER_TILE

    def chunk(j, carry):
        pltpu.sync_copy(src3.at[wid, j], idx_sc)
        pltpu.sync_copy(dst3.at[wid, j], idx_dc)
        cp_s = pltpu.async_copy(psrc.at[idx_sc], rs, sem)
        cp_d = pltpu.async_copy(pdst.at[idx_dc], rd, sem)
        cp_e = pltpu.async_copy(e_hbm.at[pl.ds(ebase + j * CHUNK, CHUNK)], eb, sem)
        cp_s.wait()
        cp_d.wait()
        cp_e.wait()

        def row1(i, carry2):
            for k in range(MEM_DIM // LANES):
                sl = pl.ds(k * LANES, LANES)
                rs[i, sl] = jnp.maximum(rs[i, sl] + rd[i, sl] + eb[i, sl], 0.0)
            return carry2

        lax.fori_loop(0, CHUNK, row1, 0)

        pltpu.sync_copy(rs, agg_sh.at[idx_dc], add=True)
        return carry

    lax.fori_loop(0, N_CHUNKS, chunk, 0)
    plsc.subcore_barrier()

    for b in range(ROWS_PER_TILE // CHUNK):
        r0 = row0 + b * CHUNK
        pltpu.sync_copy(agg_sh.at[pl.ds(r0, CHUNK)], rs)
        pltpu.sync_copy(rs, agg_out.at[c, pl.ds(r0, CHUNK)])


def _sc_aggregate(psrc, pdst, e, src3, dst3):
    mesh = plsc.VectorSubcoreMesh(core_axis_name="c", subcore_axis_name="s")
    f = pl.kernel(
        _sc_body,
        out_type=jax.ShapeDtypeStruct((NC, N_PAD, MEM_DIM), jnp.float32),
        mesh=mesh,
        scratch_types=[
            pltpu.VMEM((CHUNK,), jnp.int32),
            pltpu.VMEM((CHUNK,), jnp.int32),
            pltpu.VMEM((CHUNK, MEM_DIM), jnp.float32),
            pltpu.VMEM((CHUNK, MEM_DIM), jnp.float32),
            pltpu.VMEM((CHUNK, MEM_DIM), jnp.float32),
            pltpu.SemaphoreType.DMA,
            pltpu.VMEM_SHARED((N_PAD, MEM_DIM), jnp.float32),
        ],
    )
    return f(psrc, pdst, e, src3, dst3)


def _head_body(agg_ref, cnt_ref, mem_ref, wi_ref, wh_ref, bi_ref, bh_ref,
               wfc_ref, bfc_ref, out_ref):
    a = agg_ref[0, :N_NODES] + agg_ref[1, :N_NODES]
    agg = a / jnp.maximum(cnt_ref[...], 1.0)
    mem = mem_ref[...]
    gi = jnp.dot(agg, wi_ref[...], preferred_element_type=jnp.float32) + bi_ref[...]
    gh = jnp.dot(mem, wh_ref[...], preferred_element_type=jnp.float32) + bh_ref[...]
    r = jax.nn.sigmoid(gi[:, :MEM_DIM] + gh[:, :MEM_DIM])
    z = jax.nn.sigmoid(gi[:, MEM_DIM:2 * MEM_DIM] + gh[:, MEM_DIM:2 * MEM_DIM])
    n = jnp.tanh(gi[:, 2 * MEM_DIM:] + r * gh[:, 2 * MEM_DIM:])
    upd = (1.0 - z) * n + z * mem
    tot = jnp.sum(upd, axis=0, keepdims=True) - upd[0:1, :]
    feat = jnp.tanh(tot / (N_NODES - 1.0))
    logits = jnp.dot(feat, wfc_ref[...], preferred_element_type=jnp.float32) + bfc_ref[...]
    m = jnp.max(logits, axis=1, keepdims=True)
    ex = jnp.exp(logits - m)
    out_ref[...] = ex / jnp.sum(ex, axis=1, keepdims=True)


def _gru_head(agg2, counts, memory, wi, wh, bi, bh, wfc, bfc):
    return pl.pallas_call(
        _head_body,
        out_shape=jax.ShapeDtypeStruct((1, 2), jnp.float32),
    )(agg2, counts, memory, wi, wh, bi, bh, wfc, bfc)


def kernel(sources, destinations, timestamps, edge_features, memory,
           w_time, b_time, W_msg, b_msg, W_i, W_h, b_i, b_h, W_fc, b_fc):
    wa = W_msg[:MEM_DIM]
    wb = W_msg[MEM_DIM:2 * MEM_DIM]
    we = W_msg[2 * MEM_DIM:2 * MEM_DIM + EDGE_DIM]
    wt = W_msg[2 * MEM_DIM + EDGE_DIM:]

    psrc, pdst = _project_tables(memory, wa, wb)
    e, cnt2d = _edge_dense(
        timestamps.reshape(N_EDGES // EB, 1, EB),
        edge_features,
        destinations.reshape(N_EDGES // EB, 1, EB),
        w_time.reshape(TIME_DIM, 1),
        b_time.reshape(TIME_DIM, 1),
        we.astype(jnp.bfloat16), wt.astype(jnp.bfloat16),
        b_msg.reshape(1, MEM_DIM),
    )
    counts = cnt2d.reshape(MEM_DIM * MEM_DIM, 1)[:N_NODES]

    src3 = sources.reshape(N_TILES, N_CHUNKS, CHUNK)
    dst3 = destinations.reshape(N_TILES, N_CHUNKS, CHUNK)

    agg2 = _sc_aggregate(psrc, pdst, e, src3, dst3)

    return _gru_head(
        agg2, counts, memory, W_i, W_h,
        b_i.reshape(1, 3 * MEM_DIM), b_h.reshape(1, 3 * MEM_DIM),
        W_fc, b_fc.reshape(1, 2),
    )
